# baseline (speedup 1.0000x reference)
import jax
import jax.numpy as jnp
from jax.experimental import pallas as pl

H = 768
F = 192
E = 64
NS = 2
FS = F * 2
K = 8
T = 2048


def _router_kernel(x_ref, wr_ref, w_ref, aux_ref):
    x = x_ref[...]
    logits = jnp.dot(x, wr_ref[...], preferred_element_type=jnp.float32)
    m = jnp.max(logits, axis=-1, keepdims=True)
    ex = jnp.exp(logits - m)
    probs = ex / jnp.sum(ex, axis=-1, keepdims=True)
    p = probs
    acc = jnp.zeros_like(probs)
    ssum = jnp.zeros((T, 1), jnp.float32)
    iota = jax.lax.broadcasted_iota(jnp.int32, (T, E), 1)
    for _ in range(K):
        mk = jnp.max(p, axis=-1, keepdims=True)
        idx = jnp.min(jnp.where(p == mk, iota, E), axis=-1, keepdims=True)
        sel = iota == idx
        acc = acc + jnp.where(sel, mk, 0.0)
        ssum = ssum + mk
        p = jnp.where(sel, -1.0, p)
    weights = acc / ssum
    w_ref[...] = weights
    maskf = (weights > 0).astype(jnp.float32)
    aux_ref[...] = E * jnp.sum(
        jnp.mean(probs, axis=0, keepdims=True)
        * jnp.mean(maskf, axis=0, keepdims=True),
        axis=1, keepdims=True)


def _main_kernel(x_ref, w_ref, wg_ref, wu_ref, wd_ref,
                 wgs_ref, wus_ref, wds_ref, out_ref, cnt_ref):
    i = pl.program_id(0)

    @pl.when(i == 0)
    def _init():
        out_ref[...] = jnp.zeros_like(out_ref)
        cnt_ref[...] = jnp.zeros_like(cnt_ref)

    x = x_ref[...]

    @pl.when(i < E)
    def _expert():
        g = jnp.dot(x, wg_ref[0], preferred_element_type=jnp.float32)
        u = jnp.dot(x, wu_ref[0], preferred_element_type=jnp.float32)
        hid = jnp.where(g > 0.0, g, 0.0) * u
        cnt_ref[...] += jnp.sum((hid == 0.0).astype(jnp.float32),
                                keepdims=True).reshape(1, 1)
        sel = (jax.lax.broadcasted_iota(jnp.int32, (E, F), 0) == i)
        wb = jnp.dot(w_ref[...], sel.astype(jnp.float32),
                     preferred_element_type=jnp.float32)
        out_ref[...] += jnp.dot(hid * wb, wd_ref[0],
                                preferred_element_type=jnp.float32)

    @pl.when(i >= E)
    def _shared():
        gs = jnp.dot(x, wgs_ref[0], preferred_element_type=jnp.float32)
        us = jnp.dot(x, wus_ref[0], preferred_element_type=jnp.float32)
        hs = jnp.where(gs > 0.0, gs, 0.0) * us
        out_ref[...] += jnp.dot(hs, wds_ref[0],
                                preferred_element_type=jnp.float32)


def kernel(x, Wr, Wg, Wu, Wd, Wg_s, Wu_s, Wd_s):
    b, s, h = x.shape
    xf = x.reshape(-1, h)

    weights, aux = pl.pallas_call(
        _router_kernel,
        out_shape=(
            jax.ShapeDtypeStruct((T, E), jnp.float32),
            jax.ShapeDtypeStruct((1, 1), jnp.float32),
        ),
    )(xf, Wr)

    out, cnt = pl.pallas_call(
        _main_kernel,
        grid=(E + NS,),
        in_specs=[
            pl.BlockSpec((T, H), lambda i: (0, 0)),
            pl.BlockSpec((T, E), lambda i: (0, 0)),
            pl.BlockSpec((1, H, F), lambda i: (jnp.minimum(i, E - 1), 0, 0)),
            pl.BlockSpec((1, H, F), lambda i: (jnp.minimum(i, E - 1), 0, 0)),
            pl.BlockSpec((1, F, H), lambda i: (jnp.minimum(i, E - 1), 0, 0)),
            pl.BlockSpec((1, H, FS), lambda i: (jnp.maximum(i - E, 0), 0, 0)),
            pl.BlockSpec((1, H, FS), lambda i: (jnp.maximum(i - E, 0), 0, 0)),
            pl.BlockSpec((1, FS, H), lambda i: (jnp.maximum(i - E, 0), 0, 0)),
        ],
        out_specs=(
            pl.BlockSpec((T, H), lambda i: (0, 0)),
            pl.BlockSpec((1, 1), lambda i: (0, 0)),
        ),
        out_shape=(
            jax.ShapeDtypeStruct((T, H), jnp.float32),
            jax.ShapeDtypeStruct((1, 1), jnp.float32),
        ),
    )(xf, weights, Wg, Wu, Wd, Wg_s, Wu_s, Wd_s)

    sparsity = (cnt[0, 0] / (E * T * F)).reshape(())
    return (out.reshape(b, s, h), aux.reshape(()), sparsity)


# speedup vs baseline: 1.0090x; 1.0090x over previous
import jax
import jax.numpy as jnp
from jax.experimental import pallas as pl

H = 768
F = 192
E = 64
NS = 2
FS = F * 2
K = 8
T = 2048


def _router_kernel(x_ref, wr_ref, w_ref, aux_ref):
    x = x_ref[...]
    logits = jnp.dot(x, wr_ref[...], preferred_element_type=jnp.float32)
    m = jnp.max(logits, axis=-1, keepdims=True)
    ex = jnp.exp(logits - m)
    probs = ex / jnp.sum(ex, axis=-1, keepdims=True)
    p = probs
    acc = jnp.zeros_like(probs)
    ssum = jnp.zeros((T, 1), jnp.float32)
    iota = jax.lax.broadcasted_iota(jnp.int32, (T, E), 1)
    for _ in range(K):
        mk = jnp.max(p, axis=-1, keepdims=True)
        idx = jnp.min(jnp.where(p == mk, iota, E), axis=-1, keepdims=True)
        sel = iota == idx
        acc = acc + jnp.where(sel, mk, 0.0)
        ssum = ssum + mk
        p = jnp.where(sel, -1.0, p)
    weights = acc / ssum
    w_ref[...] = weights
    maskf = (weights > 0).astype(jnp.float32)
    aux_ref[...] = E * jnp.sum(
        jnp.mean(probs, axis=0, keepdims=True)
        * jnp.mean(maskf, axis=0, keepdims=True),
        axis=1, keepdims=True)


def _main_kernel(x_ref, w_ref, wg_ref, wu_ref, wd_ref,
                 wgs_ref, wus_ref, wds_ref, out_ref, cnt_ref):
    i = pl.program_id(0)

    @pl.when(i == 0)
    def _init():
        out_ref[...] = jnp.zeros_like(out_ref)
        cnt_ref[...] = jnp.zeros_like(cnt_ref)

    x = x_ref[...]

    @pl.when(i < E)
    def _expert():
        g = jnp.dot(x, wg_ref[0].astype(jnp.bfloat16),
                    preferred_element_type=jnp.float32)
        u = jnp.dot(x, wu_ref[0].astype(jnp.bfloat16),
                    preferred_element_type=jnp.float32)
        hid = jnp.where(g > 0.0, g, 0.0) * u
        cnt_ref[...] += jnp.sum((hid == 0.0).astype(jnp.float32),
                                keepdims=True).reshape(1, 1)
        sel = (jax.lax.broadcasted_iota(jnp.int32, (E, F), 0) == i)
        wb = jnp.dot(w_ref[...], sel.astype(jnp.float32),
                     preferred_element_type=jnp.float32)
        out_ref[...] += jnp.dot((hid * wb).astype(jnp.bfloat16),
                                wd_ref[0].astype(jnp.bfloat16),
                                preferred_element_type=jnp.float32)

    @pl.when(i >= E)
    def _shared():
        gs = jnp.dot(x, wgs_ref[0].astype(jnp.bfloat16),
                     preferred_element_type=jnp.float32)
        us = jnp.dot(x, wus_ref[0].astype(jnp.bfloat16),
                     preferred_element_type=jnp.float32)
        hs = jnp.where(gs > 0.0, gs, 0.0) * us
        out_ref[...] += jnp.dot(hs.astype(jnp.bfloat16),
                                wds_ref[0].astype(jnp.bfloat16),
                                preferred_element_type=jnp.float32)


def kernel(x, Wr, Wg, Wu, Wd, Wg_s, Wu_s, Wd_s):
    b, s, h = x.shape
    xf = x.reshape(-1, h)

    xb = xf.astype(jnp.bfloat16)

    weights, aux = pl.pallas_call(
        _router_kernel,
        out_shape=(
            jax.ShapeDtypeStruct((T, E), jnp.float32),
            jax.ShapeDtypeStruct((1, 1), jnp.float32),
        ),
    )(xf, Wr)

    out, cnt = pl.pallas_call(
        _main_kernel,
        grid=(E + NS,),
        in_specs=[
            pl.BlockSpec((T, H), lambda i: (0, 0)),
            pl.BlockSpec((T, E), lambda i: (0, 0)),
            pl.BlockSpec((1, H, F), lambda i: (jnp.minimum(i, E - 1), 0, 0)),
            pl.BlockSpec((1, H, F), lambda i: (jnp.minimum(i, E - 1), 0, 0)),
            pl.BlockSpec((1, F, H), lambda i: (jnp.minimum(i, E - 1), 0, 0)),
            pl.BlockSpec((1, H, FS), lambda i: (jnp.maximum(i - E, 0), 0, 0)),
            pl.BlockSpec((1, H, FS), lambda i: (jnp.maximum(i - E, 0), 0, 0)),
            pl.BlockSpec((1, FS, H), lambda i: (jnp.maximum(i - E, 0), 0, 0)),
        ],
        out_specs=(
            pl.BlockSpec((T, H), lambda i: (0, 0)),
            pl.BlockSpec((1, 1), lambda i: (0, 0)),
        ),
        out_shape=(
            jax.ShapeDtypeStruct((T, H), jnp.float32),
            jax.ShapeDtypeStruct((1, 1), jnp.float32),
        ),
    )(xb, weights, Wg, Wu, Wd, Wg_s, Wu_s, Wd_s)

    sparsity = (cnt[0, 0] / (E * T * F)).reshape(())
    return (out.reshape(b, s, h), aux.reshape(()), sparsity)


# speedup vs baseline: 1.1502x; 1.1400x over previous
import jax
import jax.numpy as jnp
from jax.experimental import pallas as pl

H = 768
F = 192
E = 64
NS = 2
FS = F * 2
K = 8
T = 2048
G = 4
NG = E // G


def _router_kernel(x_ref, wr_ref, w_ref, aux_ref):
    x = x_ref[...]
    logits = jnp.dot(x, wr_ref[...], preferred_element_type=jnp.float32)
    m = jnp.max(logits, axis=-1, keepdims=True)
    ex = jnp.exp(logits - m)
    probs = ex / jnp.sum(ex, axis=-1, keepdims=True)
    p = probs
    acc = jnp.zeros_like(probs)
    ssum = jnp.zeros((T, 1), jnp.float32)
    iota = jax.lax.broadcasted_iota(jnp.int32, (T, E), 1)
    for _ in range(K):
        mk = jnp.max(p, axis=-1, keepdims=True)
        idx = jnp.min(jnp.where(p == mk, iota, E), axis=-1, keepdims=True)
        sel = iota == idx
        acc = acc + jnp.where(sel, mk, 0.0)
        ssum = ssum + mk
        p = jnp.where(sel, -1.0, p)
    weights = acc / ssum
    w_ref[...] = weights
    maskf = (weights > 0).astype(jnp.float32)
    aux_ref[...] = E * jnp.sum(
        jnp.mean(probs, axis=0, keepdims=True)
        * jnp.mean(maskf, axis=0, keepdims=True),
        axis=1, keepdims=True)


def _main_kernel(x_ref, w_ref, wg_ref, wu_ref, wd_ref,
                 wgs_ref, wus_ref, wds_ref, out_ref, cnt_ref):
    i = pl.program_id(0)

    @pl.when(i == 0)
    def _init():
        out_ref[...] = jnp.zeros_like(out_ref)
        cnt_ref[...] = jnp.zeros_like(cnt_ref)

    x = x_ref[...]

    @pl.when(i < NG)
    def _experts():
        hids = []
        sels = []
        ei = jax.lax.broadcasted_iota(jnp.int32, (E, F), 0)
        for j in range(G):
            g = jnp.dot(x, wg_ref[j].astype(jnp.bfloat16),
                        preferred_element_type=jnp.float32)
            u = jnp.dot(x, wu_ref[j].astype(jnp.bfloat16),
                        preferred_element_type=jnp.float32)
            hids.append(jnp.where(g > 0.0, g, 0.0) * u)
            sels.append((ei == G * i + j).astype(jnp.float32))
        hid4 = jnp.concatenate(hids, axis=1)
        cnt_ref[...] += jnp.sum((hid4 == 0.0).astype(jnp.float32),
                                keepdims=True).reshape(1, 1)
        sel4 = jnp.concatenate(sels, axis=1)
        wb4 = jnp.dot(w_ref[...], sel4, preferred_element_type=jnp.float32)
        wd4 = wd_ref[...].reshape(G * F, H).astype(jnp.bfloat16)
        out_ref[...] += jnp.dot((hid4 * wb4).astype(jnp.bfloat16), wd4,
                                preferred_element_type=jnp.float32)

    @pl.when(i == NG)
    def _shared():
        hss = []
        for j in range(NS):
            gs = jnp.dot(x, wgs_ref[j].astype(jnp.bfloat16),
                         preferred_element_type=jnp.float32)
            us = jnp.dot(x, wus_ref[j].astype(jnp.bfloat16),
                         preferred_element_type=jnp.float32)
            hss.append(jnp.where(gs > 0.0, gs, 0.0) * us)
        hs2 = jnp.concatenate(hss, axis=1).astype(jnp.bfloat16)
        wds2 = wds_ref[...].reshape(NS * FS, H).astype(jnp.bfloat16)
        out_ref[...] += jnp.dot(hs2, wds2, preferred_element_type=jnp.float32)


def kernel(x, Wr, Wg, Wu, Wd, Wg_s, Wu_s, Wd_s):
    b, s, h = x.shape
    xf = x.reshape(-1, h)
    xb = xf.astype(jnp.bfloat16)

    weights, aux = pl.pallas_call(
        _router_kernel,
        out_shape=(
            jax.ShapeDtypeStruct((T, E), jnp.float32),
            jax.ShapeDtypeStruct((1, 1), jnp.float32),
        ),
    )(xf, Wr)

    out, cnt = pl.pallas_call(
        _main_kernel,
        grid=(NG + 1,),
        in_specs=[
            pl.BlockSpec((T, H), lambda i: (0, 0)),
            pl.BlockSpec((T, E), lambda i: (0, 0)),
            pl.BlockSpec((G, H, F), lambda i: (jnp.minimum(i, NG - 1), 0, 0)),
            pl.BlockSpec((G, H, F), lambda i: (jnp.minimum(i, NG - 1), 0, 0)),
            pl.BlockSpec((G, F, H), lambda i: (jnp.minimum(i, NG - 1), 0, 0)),
            pl.BlockSpec((NS, H, FS), lambda i: (0, 0, 0)),
            pl.BlockSpec((NS, H, FS), lambda i: (0, 0, 0)),
            pl.BlockSpec((NS, FS, H), lambda i: (0, 0, 0)),
        ],
        out_specs=(
            pl.BlockSpec((T, H), lambda i: (0, 0)),
            pl.BlockSpec((1, 1), lambda i: (0, 0)),
        ),
        out_shape=(
            jax.ShapeDtypeStruct((T, H), jnp.float32),
            jax.ShapeDtypeStruct((1, 1), jnp.float32),
        ),
    )(xb, weights, Wg, Wu, Wd, Wg_s, Wu_s, Wd_s)

    sparsity = (cnt[0, 0] / (E * T * F)).reshape(())
    return (out.reshape(b, s, h), aux.reshape(()), sparsity)


# speedup vs baseline: 1.2151x; 1.0564x over previous
import jax
import jax.numpy as jnp
from jax.experimental import pallas as pl

H = 768
F = 192
E = 64
NS = 2
FS = F * 2
K = 8
T = 2048
G = 4
NG = E // G


def _router_kernel(x_ref, wr_ref, w_ref, aux_ref):
    x = x_ref[...]
    logits = jnp.dot(x, wr_ref[...], preferred_element_type=jnp.float32)
    m = jnp.max(logits, axis=-1, keepdims=True)
    ex = jnp.exp(logits - m)
    probs = ex / jnp.sum(ex, axis=-1, keepdims=True)
    p = probs
    acc = jnp.zeros_like(probs)
    ssum = jnp.zeros((T, 1), jnp.float32)
    iota = jax.lax.broadcasted_iota(jnp.int32, (T, E), 1)
    for _ in range(K):
        mk = jnp.max(p, axis=-1, keepdims=True)
        idx = jnp.min(jnp.where(p == mk, iota, E), axis=-1, keepdims=True)
        sel = iota == idx
        acc = acc + jnp.where(sel, mk, 0.0)
        ssum = ssum + mk
        p = jnp.where(sel, -1.0, p)
    weights = acc / ssum
    w_ref[...] = weights
    maskf = (weights > 0).astype(jnp.float32)
    aux_ref[...] = E * jnp.sum(
        jnp.mean(probs, axis=0, keepdims=True)
        * jnp.mean(maskf, axis=0, keepdims=True),
        axis=1, keepdims=True)


def _main_kernel(x_ref, w_ref, wg_ref, wu_ref, wd_ref,
                 wgs_ref, wus_ref, wds_ref, out_ref, cnt_ref):
    i = pl.program_id(0)

    @pl.when(i == 0)
    def _init():
        out_ref[...] = jnp.zeros_like(out_ref)
        cnt_ref[...] = jnp.zeros_like(cnt_ref)

    x = x_ref[...]

    @pl.when(i < NG)
    def _experts():
        rhs = jnp.concatenate(
            [w_ref_j[j].astype(jnp.bfloat16)
             for j in range(G) for w_ref_j in (wg_ref, wu_ref)],
            axis=1)
        gu = jnp.dot(x, rhs, preferred_element_type=jnp.float32)
        hids = []
        sels = []
        ei = jax.lax.broadcasted_iota(jnp.int32, (E, F), 0)
        for j in range(G):
            g = gu[:, j * 2 * F: j * 2 * F + F]
            u = gu[:, j * 2 * F + F: (j + 1) * 2 * F]
            hids.append(jnp.where(g > 0.0, g, 0.0) * u)
            sels.append((ei == G * i + j).astype(jnp.float32))
        hid4 = jnp.concatenate(hids, axis=1)
        cnt_ref[...] += jnp.sum((hid4 == 0.0).astype(jnp.float32),
                                keepdims=True).reshape(1, 1)
        sel4 = jnp.concatenate(sels, axis=1)
        wb4 = jnp.dot(w_ref[...], sel4, preferred_element_type=jnp.float32)
        wd4 = wd_ref[...].reshape(G * F, H).astype(jnp.bfloat16)
        out_ref[...] += jnp.dot((hid4 * wb4).astype(jnp.bfloat16), wd4,
                                preferred_element_type=jnp.float32)

    @pl.when(i == NG)
    def _shared():
        rhs = jnp.concatenate(
            [w_ref_j[j].astype(jnp.bfloat16)
             for j in range(NS) for w_ref_j in (wgs_ref, wus_ref)],
            axis=1)
        gus = jnp.dot(x, rhs, preferred_element_type=jnp.float32)
        hss = []
        for j in range(NS):
            gs = gus[:, j * 2 * FS: j * 2 * FS + FS]
            us = gus[:, j * 2 * FS + FS: (j + 1) * 2 * FS]
            hss.append(jnp.where(gs > 0.0, gs, 0.0) * us)
        hs2 = jnp.concatenate(hss, axis=1).astype(jnp.bfloat16)
        wds2 = wds_ref[...].reshape(NS * FS, H).astype(jnp.bfloat16)
        out_ref[...] += jnp.dot(hs2, wds2, preferred_element_type=jnp.float32)


def kernel(x, Wr, Wg, Wu, Wd, Wg_s, Wu_s, Wd_s):
    b, s, h = x.shape
    xf = x.reshape(-1, h)
    xb = xf.astype(jnp.bfloat16)

    weights, aux = pl.pallas_call(
        _router_kernel,
        out_shape=(
            jax.ShapeDtypeStruct((T, E), jnp.float32),
            jax.ShapeDtypeStruct((1, 1), jnp.float32),
        ),
    )(xf, Wr)

    out, cnt = pl.pallas_call(
        _main_kernel,
        grid=(NG + 1,),
        in_specs=[
            pl.BlockSpec((T, H), lambda i: (0, 0)),
            pl.BlockSpec((T, E), lambda i: (0, 0)),
            pl.BlockSpec((G, H, F), lambda i: (jnp.minimum(i, NG - 1), 0, 0)),
            pl.BlockSpec((G, H, F), lambda i: (jnp.minimum(i, NG - 1), 0, 0)),
            pl.BlockSpec((G, F, H), lambda i: (jnp.minimum(i, NG - 1), 0, 0)),
            pl.BlockSpec((NS, H, FS), lambda i: (0, 0, 0)),
            pl.BlockSpec((NS, H, FS), lambda i: (0, 0, 0)),
            pl.BlockSpec((NS, FS, H), lambda i: (0, 0, 0)),
        ],
        out_specs=(
            pl.BlockSpec((T, H), lambda i: (0, 0)),
            pl.BlockSpec((1, 1), lambda i: (0, 0)),
        ),
        out_shape=(
            jax.ShapeDtypeStruct((T, H), jnp.float32),
            jax.ShapeDtypeStruct((1, 1), jnp.float32),
        ),
    )(xb, weights, Wg, Wu, Wd, Wg_s, Wu_s, Wd_s)

    sparsity = (cnt[0, 0] / (E * T * F)).reshape(())
    return (out.reshape(b, s, h), aux.reshape(()), sparsity)


# speedup vs baseline: 1.2426x; 1.0227x over previous
import jax
import jax.numpy as jnp
from jax.experimental import pallas as pl
from jax.experimental.pallas import tpu as pltpu

H = 768
F = 192
E = 64
NS = 2
FS = F * 2
K = 8
T = 2048
G = 4
NG = E // G


def _moe_kernel(xf_ref, wr_ref, xb_ref, wg_ref, wu_ref, wd_ref,
                wgs_ref, wus_ref, wds_ref, out_ref, aux_ref, cnt_ref, w_ref):
    i = pl.program_id(0)

    @pl.when(i == 0)
    def _router():
        out_ref[...] = jnp.zeros_like(out_ref)
        cnt_ref[...] = jnp.zeros_like(cnt_ref)
        x = xf_ref[...]
        logits = jnp.dot(x, wr_ref[...], preferred_element_type=jnp.float32)
        m = jnp.max(logits, axis=-1, keepdims=True)
        ex = jnp.exp(logits - m)
        probs = ex / jnp.sum(ex, axis=-1, keepdims=True)
        p = probs
        acc = jnp.zeros_like(probs)
        ssum = jnp.zeros((T, 1), jnp.float32)
        iota = jax.lax.broadcasted_iota(jnp.int32, (T, E), 1)
        for _ in range(K):
            mk = jnp.max(p, axis=-1, keepdims=True)
            idx = jnp.min(jnp.where(p == mk, iota, E), axis=-1, keepdims=True)
            sel = iota == idx
            acc = acc + jnp.where(sel, mk, 0.0)
            ssum = ssum + mk
            p = jnp.where(sel, -1.0, p)
        weights = acc / ssum
        w_ref[...] = weights
        maskf = (weights > 0).astype(jnp.float32)
        aux_ref[...] = E * jnp.sum(
            jnp.mean(probs, axis=0, keepdims=True)
            * jnp.mean(maskf, axis=0, keepdims=True),
            axis=1, keepdims=True)

    @pl.when((i > 0) & (i <= NG))
    def _experts():
        x = xb_ref[...]
        e0 = G * (i - 1)
        rhs = jnp.concatenate(
            [w_ref_j[j].astype(jnp.bfloat16)
             for j in range(G) for w_ref_j in (wg_ref, wu_ref)],
            axis=1)
        gu = jnp.dot(x, rhs, preferred_element_type=jnp.float32)
        hids = []
        sels = []
        ei = jax.lax.broadcasted_iota(jnp.int32, (E, F), 0)
        for j in range(G):
            g = gu[:, j * 2 * F: j * 2 * F + F]
            u = gu[:, j * 2 * F + F: (j + 1) * 2 * F]
            hids.append(jnp.where(g > 0.0, g, 0.0) * u)
            sels.append((ei == e0 + j).astype(jnp.float32))
        hid4 = jnp.concatenate(hids, axis=1)
        cnt_ref[...] += jnp.sum((hid4 == 0.0).astype(jnp.float32),
                                keepdims=True).reshape(1, 1)
        sel4 = jnp.concatenate(sels, axis=1)
        wb4 = jnp.dot(w_ref[...], sel4, preferred_element_type=jnp.float32)
        wd4 = wd_ref[...].reshape(G * F, H).astype(jnp.bfloat16)
        out_ref[...] += jnp.dot((hid4 * wb4).astype(jnp.bfloat16), wd4,
                                preferred_element_type=jnp.float32)

    @pl.when(i == NG + 1)
    def _shared():
        x = xb_ref[...]
        rhs = jnp.concatenate(
            [w_ref_j[j].astype(jnp.bfloat16)
             for j in range(NS) for w_ref_j in (wgs_ref, wus_ref)],
            axis=1)
        gus = jnp.dot(x, rhs, preferred_element_type=jnp.float32)
        hss = []
        for j in range(NS):
            gs = gus[:, j * 2 * FS: j * 2 * FS + FS]
            us = gus[:, j * 2 * FS + FS: (j + 1) * 2 * FS]
            hss.append(jnp.where(gs > 0.0, gs, 0.0) * us)
        hs2 = jnp.concatenate(hss, axis=1).astype(jnp.bfloat16)
        wds2 = wds_ref[...].reshape(NS * FS, H).astype(jnp.bfloat16)
        out_ref[...] += jnp.dot(hs2, wds2, preferred_element_type=jnp.float32)


def kernel(x, Wr, Wg, Wu, Wd, Wg_s, Wu_s, Wd_s):
    b, s, h = x.shape
    xf = x.reshape(-1, h)
    xb = xf.astype(jnp.bfloat16)

    gidx = lambda i: (jnp.clip(i - 1, 0, NG - 1), 0, 0)
    out, aux, cnt = pl.pallas_call(
        _moe_kernel,
        grid=(NG + 2,),
        in_specs=[
            pl.BlockSpec((T, H), lambda i: (0, 0)),
            pl.BlockSpec((H, E), lambda i: (0, 0)),
            pl.BlockSpec((T, H), lambda i: (0, 0)),
            pl.BlockSpec((G, H, F), gidx),
            pl.BlockSpec((G, H, F), gidx),
            pl.BlockSpec((G, F, H), gidx),
            pl.BlockSpec((NS, H, FS), lambda i: (0, 0, 0)),
            pl.BlockSpec((NS, H, FS), lambda i: (0, 0, 0)),
            pl.BlockSpec((NS, FS, H), lambda i: (0, 0, 0)),
        ],
        out_specs=(
            pl.BlockSpec((T, H), lambda i: (0, 0)),
            pl.BlockSpec((1, 1), lambda i: (0, 0)),
            pl.BlockSpec((1, 1), lambda i: (0, 0)),
        ),
        out_shape=(
            jax.ShapeDtypeStruct((T, H), jnp.float32),
            jax.ShapeDtypeStruct((1, 1), jnp.float32),
            jax.ShapeDtypeStruct((1, 1), jnp.float32),
        ),
        scratch_shapes=[pltpu.VMEM((T, E), jnp.float32)],
    )(xf, Wr, xb, Wg, Wu, Wd, Wg_s, Wu_s, Wd_s)

    sparsity = (cnt[0, 0] / (E * T * F)).reshape(())
    return (out.reshape(b, s, h), aux.reshape(()), sparsity)


# speedup vs baseline: 1.2870x; 1.0357x over previous
import jax
import jax.numpy as jnp
from jax.experimental import pallas as pl
from jax.experimental.pallas import tpu as pltpu

H = 768
F = 192
E = 64
NS = 2
FS = F * 2
K = 8
T = 2048
G = 4
NG = E // G


def _moe_kernel(xf_ref, wr_ref, xb_ref, wg_ref, wu_ref, wd_ref,
                wgs_ref, wus_ref, wds_ref, out_ref, aux_ref, cnt_ref, w_ref):
    i = pl.program_id(0)

    @pl.when(i == 0)
    def _router():
        out_ref[...] = jnp.zeros_like(out_ref)
        cnt_ref[...] = jnp.zeros_like(cnt_ref)
        x = xf_ref[...]
        logits = jnp.dot(x, wr_ref[...], preferred_element_type=jnp.float32)
        m = jnp.max(logits, axis=-1, keepdims=True)
        ex = jnp.exp(logits - m)
        probs = ex / jnp.sum(ex, axis=-1, keepdims=True)
        p = probs
        acc = jnp.zeros_like(probs)
        ssum = jnp.zeros((T, 1), jnp.float32)
        iota = jax.lax.broadcasted_iota(jnp.int32, (T, E), 1)
        for _ in range(K):
            mk = jnp.max(p, axis=-1, keepdims=True)
            idx = jnp.min(jnp.where(p == mk, iota, E), axis=-1, keepdims=True)
            sel = iota == idx
            acc = acc + jnp.where(sel, mk, 0.0)
            ssum = ssum + mk
            p = jnp.where(sel, -1.0, p)
        weights = acc / ssum
        w_ref[...] = weights
        maskf = (weights > 0).astype(jnp.float32)
        aux_ref[...] = E * jnp.sum(
            jnp.mean(probs, axis=0, keepdims=True)
            * jnp.mean(maskf, axis=0, keepdims=True),
            axis=1, keepdims=True)

    @pl.when((i > 0) & (i <= NG))
    def _experts():
        x = xb_ref[...]
        e0 = G * (i - 1)
        rhs = jnp.concatenate(
            [wg_ref[j].astype(jnp.bfloat16) for j in range(G)]
            + [wu_ref[j].astype(jnp.bfloat16) for j in range(G)],
            axis=1)
        gu = jnp.dot(x, rhs, preferred_element_type=jnp.float32)
        g4 = gu[:, :G * F]
        u4 = gu[:, G * F:]
        hid4 = jnp.where(g4 > 0.0, g4, 0.0) * u4
        cnt_ref[...] += jnp.sum((hid4 == 0.0).astype(jnp.float32),
                                keepdims=True).reshape(1, 1)
        ei = jax.lax.broadcasted_iota(jnp.int32, (E, F), 0)
        sel4 = jnp.concatenate(
            [(ei == e0 + j).astype(jnp.float32) for j in range(G)],
            axis=1)
        wb4 = jnp.dot(w_ref[...], sel4, preferred_element_type=jnp.float32)
        wd4 = wd_ref[...].reshape(G * F, H).astype(jnp.bfloat16)
        out_ref[...] += jnp.dot((hid4 * wb4).astype(jnp.bfloat16), wd4,
                                preferred_element_type=jnp.float32)

    @pl.when(i == NG + 1)
    def _shared():
        x = xb_ref[...]
        rhs = jnp.concatenate(
            [wgs_ref[j].astype(jnp.bfloat16) for j in range(NS)]
            + [wus_ref[j].astype(jnp.bfloat16) for j in range(NS)],
            axis=1)
        gus = jnp.dot(x, rhs, preferred_element_type=jnp.float32)
        gs2 = gus[:, :NS * FS]
        us2 = gus[:, NS * FS:]
        hs2 = (jnp.where(gs2 > 0.0, gs2, 0.0) * us2).astype(jnp.bfloat16)
        wds2 = wds_ref[...].reshape(NS * FS, H).astype(jnp.bfloat16)
        out_ref[...] += jnp.dot(hs2, wds2, preferred_element_type=jnp.float32)


def kernel(x, Wr, Wg, Wu, Wd, Wg_s, Wu_s, Wd_s):
    b, s, h = x.shape
    xf = x.reshape(-1, h)
    xb = xf.astype(jnp.bfloat16)

    gidx = lambda i: (jnp.clip(i - 1, 0, NG - 1), 0, 0)
    out, aux, cnt = pl.pallas_call(
        _moe_kernel,
        grid=(NG + 2,),
        in_specs=[
            pl.BlockSpec((T, H), lambda i: (0, 0)),
            pl.BlockSpec((H, E), lambda i: (0, 0)),
            pl.BlockSpec((T, H), lambda i: (0, 0)),
            pl.BlockSpec((G, H, F), gidx),
            pl.BlockSpec((G, H, F), gidx),
            pl.BlockSpec((G, F, H), gidx),
            pl.BlockSpec((NS, H, FS), lambda i: (0, 0, 0)),
            pl.BlockSpec((NS, H, FS), lambda i: (0, 0, 0)),
            pl.BlockSpec((NS, FS, H), lambda i: (0, 0, 0)),
        ],
        out_specs=(
            pl.BlockSpec((T, H), lambda i: (0, 0)),
            pl.BlockSpec((1, 1), lambda i: (0, 0)),
            pl.BlockSpec((1, 1), lambda i: (0, 0)),
        ),
        out_shape=(
            jax.ShapeDtypeStruct((T, H), jnp.float32),
            jax.ShapeDtypeStruct((1, 1), jnp.float32),
            jax.ShapeDtypeStruct((1, 1), jnp.float32),
        ),
        scratch_shapes=[pltpu.VMEM((T, E), jnp.float32)],
    )(xf, Wr, xb, Wg, Wu, Wd, Wg_s, Wu_s, Wd_s)

    sparsity = (cnt[0, 0] / (E * T * F)).reshape(())
    return (out.reshape(b, s, h), aux.reshape(()), sparsity)
